# Initial kernel scaffold; baseline (speedup 1.0000x reference)
import jax
import jax.numpy as jnp
from jax.experimental import pallas as pl


def kernel(positions, embeddings):
    raise NotImplementedError("write your pallas kernel here")


# speedup vs baseline: 2.3845x; 2.3845x over previous
import functools

import jax
import jax.numpy as jnp
from jax import lax
from jax.experimental import pallas as pl
from jax.experimental.pallas import tpu as pltpu
from jax.experimental.pallas import tpu_sc as plsc

N_POS = 8192
DIM = 768
N_IDX = 4 * 8192
NUM_CORES = 2
NUM_SUBCORES = 16
NW = NUM_CORES * NUM_SUBCORES
BPW = N_IDX // NW
CHUNK = 128
NCHUNK = BPW // CHUNK


@functools.partial(
    pl.kernel,
    mesh=plsc.VectorSubcoreMesh(core_axis_name="c", subcore_axis_name="s"),
    out_type=jax.ShapeDtypeStruct((N_IDX, DIM), jnp.float32),
    scratch_types=[
        pltpu.VMEM((NCHUNK, CHUNK), jnp.int32),
        pltpu.VMEM((CHUNK, DIM), jnp.float32),
        pltpu.SemaphoreType.DMA,
    ],
)
def _gather_kernel(idx_hbm, table_hbm, out_hbm, idx_v, rows_v, sem):
    wid = lax.axis_index("s") * NUM_CORES + lax.axis_index("c")
    base = wid * BPW
    pltpu.sync_copy(idx_hbm.at[wid], idx_v)
    for i in range(NCHUNK):
        pltpu.async_copy(table_hbm.at[idx_v.at[i]], rows_v, sem).wait()
        pltpu.sync_copy(rows_v, out_hbm.at[pl.ds(base + i * CHUNK, CHUNK)])


def kernel(positions, embeddings):
    idx = positions.reshape(NW, NCHUNK, CHUNK)
    out = _gather_kernel(idx, embeddings)
    return out.reshape(positions.shape + (DIM,))


# speedup vs baseline: 2.4334x; 1.0205x over previous
import functools

import jax
import jax.numpy as jnp
from jax import lax
from jax.experimental import pallas as pl
from jax.experimental.pallas import tpu as pltpu
from jax.experimental.pallas import tpu_sc as plsc

N_POS = 8192
DIM = 768
N_IDX = 4 * 8192
NUM_CORES = 2
NUM_SUBCORES = 16
NW = NUM_CORES * NUM_SUBCORES
BPW = N_IDX // NW
CHUNK = 64
NCHUNK = BPW // CHUNK


@functools.partial(
    pl.kernel,
    mesh=plsc.VectorSubcoreMesh(core_axis_name="c", subcore_axis_name="s"),
    out_type=jax.ShapeDtypeStruct((N_IDX, DIM), jnp.float32),
    scratch_types=[
        pltpu.VMEM((NCHUNK, CHUNK), jnp.int32),
        pltpu.VMEM((CHUNK, DIM), jnp.float32),
        pltpu.VMEM((CHUNK, DIM), jnp.float32),
        pltpu.SemaphoreType.DMA,
        pltpu.SemaphoreType.DMA,
    ],
)
def _gather_kernel(idx_hbm, table_hbm, out_hbm, idx_v, buf0, buf1, gsem, osem):
    wid = lax.axis_index("s") * NUM_CORES + lax.axis_index("c")
    base = wid * BPW
    bufs = (buf0, buf1)
    pltpu.sync_copy(idx_hbm.at[wid], idx_v)
    pltpu.async_copy(table_hbm.at[idx_v.at[0]], bufs[0], gsem)
    for i in range(NCHUNK):
        buf = bufs[i % 2]
        gwait = pltpu.make_async_copy(table_hbm.at[idx_v.at[i]], buf, gsem)
        gwait.wait()
        if i >= 1:
            prev = bufs[(i - 1) % 2]
            pltpu.make_async_copy(
                prev, out_hbm.at[pl.ds(base + (i - 1) * CHUNK, CHUNK)], osem
            ).wait()
        if i + 1 < NCHUNK:
            pltpu.async_copy(table_hbm.at[idx_v.at[i + 1]], bufs[(i + 1) % 2], gsem)
        pltpu.async_copy(buf, out_hbm.at[pl.ds(base + i * CHUNK, CHUNK)], osem)
    pltpu.make_async_copy(
        bufs[(NCHUNK - 1) % 2],
        out_hbm.at[pl.ds(base + (NCHUNK - 1) * CHUNK, CHUNK)],
        osem,
    ).wait()


def kernel(positions, embeddings):
    idx = positions.reshape(NW, NCHUNK, CHUNK)
    out = _gather_kernel(idx, embeddings)
    return out.reshape(positions.shape + (DIM,))
